# baseline (speedup 1.0000x reference)
import functools

import jax
import jax.numpy as jnp
from jax import lax
from jax.experimental import pallas as pl
from jax.experimental.pallas import tpu as pltpu
from jax.experimental.pallas import tpu_sc as plsc

B = 16384
HID = 64
NROWS = 1000000
NPAIR = NROWS // 2

_NC, _NS = 2, 16
_NW = _NC * _NS
_BPW = B // _NW
_CH = 64
_NCHUNK = _BPW // _CH


def _splat_lane(v, j):
    dn = lax.GatherDimensionNumbers(
        offset_dims=(), collapsed_slice_dims=(0,), start_index_map=(0,))
    idx = jnp.full((16, 1), j, jnp.int32)
    return lax.gather(v, idx, dn, (1,),
                      mode=lax.GatherScatterMode.PROMISE_IN_BOUNDS)


def _gather3_body(u_idx, p_idx, n_idx, utab, itab, ue, pe, ne,
                  idx_v, pair_v, buf, rows, sem):
    wid = lax.axis_index("s") * _NC + lax.axis_index("c")
    base = wid * _BPW
    iota = lax.iota(jnp.int32, 16)

    for idx_hbm, tab, out in ((u_idx, utab, ue), (p_idx, itab, pe),
                              (n_idx, itab, ne)):
        pltpu.sync_copy(idx_hbm.at[pl.ds(base, _BPW)], idx_v)

        def pre(i, _):
            off = pl.multiple_of(i * 16, 16)
            v = idx_v[pl.ds(off, 16)]
            pair_v[pl.ds(off, 16)] = jnp.right_shift(v, 1)
            return 0

        lax.fori_loop(0, _BPW // 16, pre, 0, unroll=8)

        def chunk(c, _):
            cb = pl.multiple_of(c * _CH, _CH)
            pltpu.async_copy(
                tab.at[pair_v.at[pl.ds(cb, _CH)]], buf, sem).wait()
            for g in range(_CH // 16):
                v = idx_v[pl.ds(cb + g * 16, 16)]
                half16 = jnp.bitwise_and(v, 1) * HID
                for r16 in range(16):
                    r = g * 16 + r16
                    h_spl = _splat_lane(half16, r16)
                    i0 = jnp.full((16,), r, jnp.int32)
                    for k in range(HID // 16):
                        got = plsc.load_gather(
                            buf, [i0, h_spl + (k * 16) + iota])
                        rows[r, pl.ds(k * 16, 16)] = got
            pltpu.sync_copy(rows, out.at[pl.ds(base + cb, _CH)])
            return 0

        lax.fori_loop(0, _NCHUNK, chunk, 0)


_GATHER3_CACHE = []


def _gather3(*args):
    if not _GATHER3_CACHE:
        _GATHER3_CACHE.append(functools.partial(
            pl.kernel,
            mesh=plsc.VectorSubcoreMesh(
                core_axis_name="c", subcore_axis_name="s",
                num_cores=_NC, num_subcores=_NS),
            out_type=[jax.ShapeDtypeStruct((B, HID), jnp.float32)] * 3,
            scratch_types=[
                pltpu.VMEM((_BPW,), jnp.int32),
                pltpu.VMEM((_BPW,), jnp.int32),
                pltpu.VMEM((_CH, 2 * HID), jnp.float32),
                pltpu.VMEM((_CH, HID), jnp.float32),
                pltpu.SemaphoreType.DMA,
            ],
            compiler_params=pltpu.CompilerParams(needs_layout_passes=False),
        )(_gather3_body))
    return _GATHER3_CACHE[0](*args)


def _dense_body(ue, pe, ne, W1, b1, W2, b2, ps, ns, cat, u_out):
    m = 0.5 * (ue[...] + pe[...])
    h = jnp.maximum(
        jnp.dot(m, W1[...], preferred_element_type=jnp.float32) + b1[...], 0.0)
    u = jnp.dot(h, W2[...], preferred_element_type=jnp.float32) + b2[...]
    hn = jnp.maximum(
        jnp.dot(ne[...], W1[...], preferred_element_type=jnp.float32) + b1[...],
        0.0)
    neg = jnp.dot(hn, W2[...], preferred_element_type=jnp.float32) + b2[...]
    ps[...] = jnp.sum(u * u, axis=1, keepdims=True)
    ns[...] = jnp.sum(u * neg, axis=1, keepdims=True)
    cat[:, 0:HID] = u
    cat[:, HID:2 * HID] = neg
    u_out[...] = u


def _dense(ue, pe, ne, W1, b1, W2, b2, bs=1024):
    grid = (B // bs,)
    row_spec = pl.BlockSpec((bs, HID), lambda i: (i, 0))
    full = pl.BlockSpec((HID, HID), lambda i: (0, 0))
    bias = pl.BlockSpec((1, HID), lambda i: (0, 0))
    return pl.pallas_call(
        _dense_body,
        grid=grid,
        in_specs=[row_spec, row_spec, row_spec, full, bias, full, bias],
        out_specs=[
            pl.BlockSpec((bs, 1), lambda i: (i, 0)),
            pl.BlockSpec((bs, 1), lambda i: (i, 0)),
            pl.BlockSpec((bs, 2 * HID), lambda i: (i, 0)),
            row_spec,
        ],
        out_shape=[
            jax.ShapeDtypeStruct((B, 1), jnp.float32),
            jax.ShapeDtypeStruct((B, 1), jnp.float32),
            jax.ShapeDtypeStruct((B, 2 * HID), jnp.float32),
            jax.ShapeDtypeStruct((B, HID), jnp.float32),
        ],
    )(ue, pe, ne, W1, b1, W2, b2)


def kernel(user, pos_item, neg_item, user_table, item_table, W1, b1, W2, b2):
    ut2 = user_table.reshape(NPAIR, 2 * HID)
    it2 = item_table.reshape(NPAIR, 2 * HID)
    ue, pe, ne = _gather3(user, pos_item, neg_item, ut2, it2)
    ps, ns, cat, u = _dense(ue, pe, ne, W1, b1.reshape(1, HID),
                            W2, b2.reshape(1, HID))
    return (ps, ns, cat, u)


# speedup vs baseline: 1.5589x; 1.5589x over previous
import functools

import jax
import jax.numpy as jnp
from jax import lax
from jax.experimental import pallas as pl
from jax.experimental.pallas import tpu as pltpu
from jax.experimental.pallas import tpu_sc as plsc

B = 16384
HID = 64
NROWS = 1000000
SUB = 8
NBLK = NROWS // SUB

_NC, _NS = 2, 16
_NW = _NC * _NS
_BPW = B // _NW
_CH = 64
_NCHUNK = _BPW // _CH


def _splat_lane(v, j):
    dn = lax.GatherDimensionNumbers(
        offset_dims=(), collapsed_slice_dims=(0,), start_index_map=(0,))
    idx = jnp.full((16, 1), j, jnp.int32)
    return lax.gather(v, idx, dn, (1,),
                      mode=lax.GatherScatterMode.PROMISE_IN_BOUNDS)


def _gather3_body(u_idx, p_idx, n_idx, utab, itab, ue, pe, ne,
                  idx_v, rows, sem):
    wid = lax.axis_index("s") * _NC + lax.axis_index("c")
    base = wid * _BPW
    iota = lax.iota(jnp.int32, 16)

    for idx_hbm, tab, out in ((u_idx, utab, ue), (p_idx, itab, pe),
                              (n_idx, itab, ne)):
        pltpu.sync_copy(idx_hbm.at[pl.ds(base, _BPW)], idx_v)

        def chunk(c, _):
            cb = pl.multiple_of(c * _CH, _CH)
            cps = []
            for g in range(_CH // 16):
                v = idx_v[pl.ds(cb + g * 16, 16)]
                scalars = [jnp.max(jnp.where(iota == r16, v, 0))
                           for r16 in range(16)]
                for r16, i in enumerate(scalars):
                    r = g * 16 + r16
                    cps.append(pltpu.async_copy(
                        tab.at[pl.ds(i, 1)], rows.at[pl.ds(r, 1)], sem))
            for cp in cps:
                cp.wait()
            pltpu.sync_copy(rows, out.at[pl.ds(base + cb, _CH)])
            return 0

        lax.fori_loop(0, _NCHUNK, chunk, 0)


_GATHER3_CACHE = []


def _gather3(*args):
    if not _GATHER3_CACHE:
        _GATHER3_CACHE.append(functools.partial(
            pl.kernel,
            mesh=plsc.VectorSubcoreMesh(
                core_axis_name="c", subcore_axis_name="s",
                num_cores=_NC, num_subcores=_NS),
            out_type=[jax.ShapeDtypeStruct((B, HID), jnp.float32)] * 3,
            scratch_types=[
                pltpu.VMEM((_BPW,), jnp.int32),
                pltpu.VMEM((_CH, HID), jnp.float32),
                pltpu.SemaphoreType.DMA,
            ],
            compiler_params=pltpu.CompilerParams(needs_layout_passes=False),
        )(_gather3_body))
    return _GATHER3_CACHE[0](*args)


def _dense_body(ue, pe, ne, W1, b1, W2, b2, ps, ns, cat, u_out):
    m = 0.5 * (ue[...] + pe[...])
    h = jnp.maximum(
        jnp.dot(m, W1[...], preferred_element_type=jnp.float32) + b1[...], 0.0)
    u = jnp.dot(h, W2[...], preferred_element_type=jnp.float32) + b2[...]
    hn = jnp.maximum(
        jnp.dot(ne[...], W1[...], preferred_element_type=jnp.float32) + b1[...],
        0.0)
    neg = jnp.dot(hn, W2[...], preferred_element_type=jnp.float32) + b2[...]
    ps[...] = jnp.sum(u * u, axis=1, keepdims=True)
    ns[...] = jnp.sum(u * neg, axis=1, keepdims=True)
    cat[:, 0:HID] = u
    cat[:, HID:2 * HID] = neg
    u_out[...] = u


def _dense(ue, pe, ne, W1, b1, W2, b2, bs=1024):
    grid = (B // bs,)
    row_spec = pl.BlockSpec((bs, HID), lambda i: (i, 0))
    full = pl.BlockSpec((HID, HID), lambda i: (0, 0))
    bias = pl.BlockSpec((1, HID), lambda i: (0, 0))
    return pl.pallas_call(
        _dense_body,
        grid=grid,
        in_specs=[row_spec, row_spec, row_spec, full, bias, full, bias],
        out_specs=[
            pl.BlockSpec((bs, 1), lambda i: (i, 0)),
            pl.BlockSpec((bs, 1), lambda i: (i, 0)),
            pl.BlockSpec((bs, 2 * HID), lambda i: (i, 0)),
            row_spec,
        ],
        out_shape=[
            jax.ShapeDtypeStruct((B, 1), jnp.float32),
            jax.ShapeDtypeStruct((B, 1), jnp.float32),
            jax.ShapeDtypeStruct((B, 2 * HID), jnp.float32),
            jax.ShapeDtypeStruct((B, HID), jnp.float32),
        ],
    )(ue, pe, ne, W1, b1, W2, b2)


def kernel(user, pos_item, neg_item, user_table, item_table, W1, b1, W2, b2):
    ue, pe, ne = _gather3(user, pos_item, neg_item, user_table, item_table)
    ps, ns, cat, u = _dense(ue, pe, ne, W1, b1.reshape(1, HID),
                            W2, b2.reshape(1, HID))
    return (ps, ns, cat, u)


# speedup vs baseline: 2.2382x; 1.4357x over previous
import functools

import jax
import jax.numpy as jnp
from jax import lax
from jax.experimental import pallas as pl
from jax.experimental.pallas import tpu as pltpu
from jax.experimental.pallas import tpu_sc as plsc

B = 16384
HID = 64
NROWS = 1000000
SUB = 8
NBLK = NROWS // SUB

_NC, _NS = 2, 16
_NW = _NC * _NS
_BPW = B // _NW
_CH = 64
_NCHUNK = _BPW // _CH


def _splat_lane(v, j):
    dn = lax.GatherDimensionNumbers(
        offset_dims=(), collapsed_slice_dims=(0,), start_index_map=(0,))
    idx = jnp.full((16, 1), j, jnp.int32)
    return lax.gather(v, idx, dn, (1,),
                      mode=lax.GatherScatterMode.PROMISE_IN_BOUNDS)


def _gather3_body(u_idx, p_idx, n_idx, utab, itab, ue, pe, ne,
                  idx_v, rows, sem):
    wid = lax.axis_index("s") * _NC + lax.axis_index("c")
    base = wid * _BPW
    iota = lax.iota(jnp.int32, 16)

    for idx_hbm, tab, out in ((u_idx, utab, ue), (p_idx, itab, pe),
                              (n_idx, itab, ne)):
        pltpu.sync_copy(idx_hbm.at[pl.ds(base, _BPW)], idx_v)

        def chunk(c, _):
            cb = pl.multiple_of(c * _CH, _CH)
            cps = []
            for g in range(_CH // 16):
                v = idx_v[pl.ds(cb + g * 16, 16)]
                blk16 = jnp.right_shift(v, 3)
                sub16 = jnp.bitwise_and(v, 7)
                for r16 in range(16):
                    r = g * 16 + r16
                    blk = jnp.max(jnp.where(iota == r16, blk16, 0))
                    sub = jnp.max(jnp.where(iota == r16, sub16, 0))
                    cps.append(
                        pltpu.async_copy(tab.at[blk, sub], rows.at[r], sem))
            for cp in cps:
                cp.wait()
            pltpu.sync_copy(rows, out.at[pl.ds(base + cb, _CH)])
            return 0

        lax.fori_loop(0, _NCHUNK, chunk, 0)


_GATHER3_CACHE = []


def _gather3(*args):
    if not _GATHER3_CACHE:
        _GATHER3_CACHE.append(functools.partial(
            pl.kernel,
            mesh=plsc.VectorSubcoreMesh(
                core_axis_name="c", subcore_axis_name="s",
                num_cores=_NC, num_subcores=_NS),
            out_type=[jax.ShapeDtypeStruct((B, HID), jnp.float32)] * 3,
            scratch_types=[
                pltpu.VMEM((_BPW,), jnp.int32),
                pltpu.VMEM((_CH, HID), jnp.float32),
                pltpu.SemaphoreType.DMA,
            ],
            compiler_params=pltpu.CompilerParams(needs_layout_passes=False),
        )(_gather3_body))
    return _GATHER3_CACHE[0](*args)


def _dense_body(ue, pe, ne, W1, b1, W2, b2, ps, ns, cat, u_out):
    m = 0.5 * (ue[...] + pe[...])
    h = jnp.maximum(
        jnp.dot(m, W1[...], preferred_element_type=jnp.float32) + b1[...], 0.0)
    u = jnp.dot(h, W2[...], preferred_element_type=jnp.float32) + b2[...]
    hn = jnp.maximum(
        jnp.dot(ne[...], W1[...], preferred_element_type=jnp.float32) + b1[...],
        0.0)
    neg = jnp.dot(hn, W2[...], preferred_element_type=jnp.float32) + b2[...]
    ps[...] = jnp.sum(u * u, axis=1, keepdims=True)
    ns[...] = jnp.sum(u * neg, axis=1, keepdims=True)
    cat[:, 0:HID] = u
    cat[:, HID:2 * HID] = neg
    u_out[...] = u


def _dense(ue, pe, ne, W1, b1, W2, b2, bs=1024):
    grid = (B // bs,)
    row_spec = pl.BlockSpec((bs, HID), lambda i: (i, 0))
    full = pl.BlockSpec((HID, HID), lambda i: (0, 0))
    bias = pl.BlockSpec((1, HID), lambda i: (0, 0))
    return pl.pallas_call(
        _dense_body,
        grid=grid,
        in_specs=[row_spec, row_spec, row_spec, full, bias, full, bias],
        out_specs=[
            pl.BlockSpec((bs, 1), lambda i: (i, 0)),
            pl.BlockSpec((bs, 1), lambda i: (i, 0)),
            pl.BlockSpec((bs, 2 * HID), lambda i: (i, 0)),
            row_spec,
        ],
        out_shape=[
            jax.ShapeDtypeStruct((B, 1), jnp.float32),
            jax.ShapeDtypeStruct((B, 1), jnp.float32),
            jax.ShapeDtypeStruct((B, 2 * HID), jnp.float32),
            jax.ShapeDtypeStruct((B, HID), jnp.float32),
        ],
    )(ue, pe, ne, W1, b1, W2, b2)


def kernel(user, pos_item, neg_item, user_table, item_table, W1, b1, W2, b2):
    ut3 = user_table.reshape(NBLK, SUB, HID)
    it3 = item_table.reshape(NBLK, SUB, HID)
    ue, pe, ne = _gather3(user, pos_item, neg_item, ut3, it3)
    ps, ns, cat, u = _dense(ue, pe, ne, W1, b1.reshape(1, HID),
                            W2, b2.reshape(1, HID))
    return (ps, ns, cat, u)


# speedup vs baseline: 2.3307x; 1.0413x over previous
import functools

import jax
import jax.numpy as jnp
from jax import lax
from jax.experimental import pallas as pl
from jax.experimental.pallas import tpu as pltpu
from jax.experimental.pallas import tpu_sc as plsc

B = 16384
HID = 64
NROWS = 1000000
SUB = 8
NBLK = NROWS // SUB

_NC, _NS = 2, 16
_NW = _NC * _NS
_BPW = B // _NW
_CH = 64
_NCHUNK = _BPW // _CH


def _splat_lane(v, j):
    dn = lax.GatherDimensionNumbers(
        offset_dims=(), collapsed_slice_dims=(0,), start_index_map=(0,))
    idx = jnp.full((16, 1), j, jnp.int32)
    return lax.gather(v, idx, dn, (1,),
                      mode=lax.GatherScatterMode.PROMISE_IN_BOUNDS)


def _gather3_body(u_idx, p_idx, n_idx, utab, itab, ue, pe, ne,
                  idx_v, rows, sem):
    wid = lax.axis_index("s") * _NC + lax.axis_index("c")
    base = wid * _BPW
    iota = lax.iota(jnp.int32, 16)

    for idx_hbm, tab, out in ((u_idx, utab, ue), (p_idx, itab, pe),
                              (n_idx, itab, ne)):
        pltpu.sync_copy(idx_hbm.at[pl.ds(base, _BPW)], idx_v)

        def chunk(c, _):
            cb = pl.multiple_of(c * _CH, _CH)
            cps = []
            for g in range(_CH // 16):
                v = idx_v[pl.ds(cb + g * 16, 16)]
                blk16 = jnp.right_shift(v, 3)
                sub16 = jnp.bitwise_and(v, 7)
                for r16 in range(16):
                    r = g * 16 + r16
                    blk = jnp.max(jnp.where(iota == r16, blk16, 0))
                    sub = jnp.max(jnp.where(iota == r16, sub16, 0))
                    cps.append(
                        pltpu.async_copy(tab.at[blk, sub], rows.at[r], sem))
            for cp in cps:
                cp.wait()
            pltpu.sync_copy(rows, out.at[pl.ds(base + cb, _CH)])
            return 0

        lax.fori_loop(0, _NCHUNK, chunk, 0)


_GATHER3_CACHE = []


def _gather3(*args):
    if not _GATHER3_CACHE:
        _GATHER3_CACHE.append(functools.partial(
            pl.kernel,
            mesh=plsc.VectorSubcoreMesh(
                core_axis_name="c", subcore_axis_name="s",
                num_cores=_NC, num_subcores=_NS),
            out_type=[jax.ShapeDtypeStruct((B, HID), jnp.float32)] * 3,
            scratch_types=[
                pltpu.VMEM((_BPW,), jnp.int32),
                pltpu.VMEM((_CH, HID), jnp.float32),
                pltpu.SemaphoreType.DMA,
            ],
            compiler_params=pltpu.CompilerParams(needs_layout_passes=False),
        )(_gather3_body))
    return _GATHER3_CACHE[0](*args)


def _dense_body(ue, pe, ne, W1, b1, W2, b2, ps, ns, catT, uT_out):
    m = 0.5 * (ue[...] + pe[...])
    h = jnp.maximum(
        jnp.dot(m, W1[...], preferred_element_type=jnp.float32) + b1[...], 0.0)
    u = jnp.dot(h, W2[...], preferred_element_type=jnp.float32) + b2[...]
    hn = jnp.maximum(
        jnp.dot(ne[...], W1[...], preferred_element_type=jnp.float32) + b1[...],
        0.0)
    neg = jnp.dot(hn, W2[...], preferred_element_type=jnp.float32) + b2[...]
    ps[...] = jnp.sum(u * u, axis=1, keepdims=True)
    ns[...] = jnp.sum(u * neg, axis=1, keepdims=True)
    uT = jnp.transpose(u)
    catT[0:HID, :] = uT
    catT[HID:2 * HID, :] = jnp.transpose(neg)
    uT_out[...] = uT


def _dense(ue, pe, ne, W1, b1, W2, b2, bs=2048):
    grid = (B // bs,)
    row_spec = pl.BlockSpec((bs, HID), lambda i: (i, 0))
    full = pl.BlockSpec((HID, HID), lambda i: (0, 0))
    bias = pl.BlockSpec((1, HID), lambda i: (0, 0))
    return pl.pallas_call(
        _dense_body,
        grid=grid,
        in_specs=[row_spec, row_spec, row_spec, full, bias, full, bias],
        out_specs=[
            pl.BlockSpec((bs, 1), lambda i: (i, 0)),
            pl.BlockSpec((bs, 1), lambda i: (i, 0)),
            pl.BlockSpec((2 * HID, bs), lambda i: (0, i)),
            pl.BlockSpec((HID, bs), lambda i: (0, i)),
        ],
        out_shape=[
            jax.ShapeDtypeStruct((B, 1), jnp.float32),
            jax.ShapeDtypeStruct((B, 1), jnp.float32),
            jax.ShapeDtypeStruct((2 * HID, B), jnp.float32),
            jax.ShapeDtypeStruct((HID, B), jnp.float32),
        ],
    )(ue, pe, ne, W1, b1, W2, b2)


def kernel(user, pos_item, neg_item, user_table, item_table, W1, b1, W2, b2):
    ut3 = user_table.reshape(NBLK, SUB, HID)
    it3 = item_table.reshape(NBLK, SUB, HID)
    ue, pe, ne = _gather3(user, pos_item, neg_item, ut3, it3)
    ps, ns, catT, uT = _dense(ue, pe, ne, W1, b1.reshape(1, HID),
                              W2, b2.reshape(1, HID))
    return (ps, ns, catT.T, uT.T)
